# baseline (speedup 1.0000x reference)
import functools

import jax
import jax.numpy as jnp
from jax import lax
from jax.experimental import pallas as pl
from jax.experimental.pallas import tpu as pltpu
from jax.experimental.pallas import tpu_sc as plsc

N_NODES = 10000
N_EDGES = 160000
D_FEAT = 256
D_OUT = 256

NC = 2
NS = 16
H = D_FEAT // 2
CHUNK = 64
NCHUNK = 160
NBUF = 4
NSTAGE = 5
EP = NS * NCHUNK * CHUNK
ACC_ROWS = 10240
RPT = ACC_ROWS // NS


def _sc_segment_sum(gidx, didx, feat2):
    mesh = plsc.VectorSubcoreMesh(core_axis_name="c", subcore_axis_name="s")

    @functools.partial(
        pl.kernel,
        out_type=jax.ShapeDtypeStruct((NC, ACC_ROWS, H), jnp.float32),
        mesh=mesh,
        scratch_types=[
            pltpu.VMEM((2 * (NCHUNK // NSTAGE), CHUNK), jnp.int32),
            pltpu.VMEM((2 * (NCHUNK // NSTAGE), CHUNK), jnp.int32),
            pltpu.VMEM((NBUF, CHUNK, H), jnp.float32),
            pltpu.VMEM_SHARED((ACC_ROWS, H), jnp.float32),
        ] + [pltpu.SemaphoreType.DMA] * (NBUF + 1),
    )
    def k(gidx_hbm, didx_hbm, feat2_hbm, out_hbm,
          gidx_v, didx_v, rows_v, acc, *sems):
        c = lax.axis_index("c")
        s = lax.axis_index("s")
        @plsc.parallel_loop(0, CHUNK, 1)
        def _(r):
            for kk in range(H // 16):
                rows_v[0, r, pl.ds(kk * 16, 16)] = jnp.zeros(
                    (16,), jnp.float32)

        for z in range(RPT // CHUNK):
            pltpu.sync_copy(rows_v.at[0],
                            acc.at[pl.ds(s * RPT + z * CHUNK, CHUNK)])
        plsc.subcore_barrier()

        HALF = NCHUNK // NSTAGE
        sem_idx = sems[NBUF]

        def gather(j, b, sl):
            return pltpu.make_async_copy(
                feat2_hbm.at[gidx_v.at[sl * HALF + j]], rows_v.at[b], sems[b])

        def stage_copies(hh, sl):
            return (
                pltpu.make_async_copy(
                    gidx_hbm.at[c, s, pl.ds(hh * HALF, HALF)],
                    gidx_v.at[pl.ds(sl * HALF, HALF)], sem_idx),
                pltpu.make_async_copy(
                    didx_hbm.at[s, pl.ds(hh * HALF, HALF)],
                    didx_v.at[pl.ds(sl * HALF, HALF)], sem_idx),
            )

        for cp in stage_copies(0, 0):
            cp.start()
        for cp in stage_copies(0, 0):
            cp.wait()
        for hh in range(NSTAGE):
            sl = hh % 2
            if hh + 1 < NSTAGE:
                for cp in stage_copies(hh + 1, 1 - sl):
                    cp.start()
            for b in range(NBUF):
                gather(b, b, sl).start()

            def body(i, _):
                for b in range(NBUF):
                    j = NBUF * i + b
                    gather(j, b, sl).wait()
                    pltpu.sync_copy(rows_v.at[b],
                                    acc.at[didx_v.at[sl * HALF + j]],
                                    add=True)
                    jn = j + NBUF

                    @pl.when(jn < HALF)
                    def _():
                        gather(jn, b, sl).start()
                return 0

            lax.fori_loop(0, HALF // NBUF, body, 0)
            if hh + 1 < NSTAGE:
                for cp in stage_copies(hh + 1, 1 - sl):
                    cp.wait()
        plsc.subcore_barrier()
        pltpu.sync_copy(acc.at[pl.ds(s * RPT, RPT)],
                        out_hbm.at[c, pl.ds(s * RPT, RPT)])

    return k(gidx, didx, feat2)


def _matmul_kernel(h_ref, w_ref, b_ref, out_ref):
    h0 = h_ref[0]
    h1 = h_ref[1]
    acc = jnp.dot(h0, w_ref[:H, :], preferred_element_type=jnp.float32)
    acc += jnp.dot(h1, w_ref[H:, :], preferred_element_type=jnp.float32)
    out_ref[...] = acc + b_ref[...]


def _tc_linear(h2, W, b):
    R = 2000
    return pl.pallas_call(
        _matmul_kernel,
        grid=(N_NODES // R,),
        in_specs=[
            pl.BlockSpec((NC, R, H), lambda i: (0, i, 0)),
            pl.BlockSpec((D_FEAT, D_OUT), lambda i: (0, 0)),
            pl.BlockSpec((1, D_OUT), lambda i: (0, 0)),
        ],
        out_specs=pl.BlockSpec((R, D_OUT), lambda i: (i, 0)),
        out_shape=jax.ShapeDtypeStruct((N_NODES, D_OUT), jnp.float32),
    )(h2, W, b.reshape(1, D_OUT))


def kernel(feat, edge_index, W, b):
    src = edge_index[0].astype(jnp.int32)
    dst = edge_index[1].astype(jnp.int32)
    src_p = jnp.zeros((EP,), jnp.int32).at[:N_EDGES].set(src)
    dst_p = jnp.full((EP,), N_NODES, jnp.int32).at[:N_EDGES].set(dst)
    gidx = (2 * src_p)[None, :] + jnp.arange(NC, dtype=jnp.int32)[:, None]
    gidx = gidx.reshape(NC, NS, NCHUNK, CHUNK)
    didx = dst_p.reshape(NS, NCHUNK, CHUNK)
    feat2 = feat.reshape(2 * N_NODES, H)

    h2 = _sc_segment_sum(gidx, didx, feat2)
    return _tc_linear(h2, W, b)


# speedup vs baseline: 1.0028x; 1.0028x over previous
import functools

import jax
import jax.numpy as jnp
from jax import lax
from jax.experimental import pallas as pl
from jax.experimental.pallas import tpu as pltpu
from jax.experimental.pallas import tpu_sc as plsc

N_NODES = 10000
N_EDGES = 160000
D_FEAT = 256
D_OUT = 256

NC = 2
NS = 16
H = D_FEAT // 2
CHUNK = 64
NCHUNK = 160
NBUF = 4
NSTAGE = 5
EP = NS * NCHUNK * CHUNK
ACC_ROWS = 10240
RPT = ACC_ROWS // NS


def _sc_segment_sum(gidx, didx, feat2):
    mesh = plsc.VectorSubcoreMesh(core_axis_name="c", subcore_axis_name="s")

    @functools.partial(
        pl.kernel,
        out_type=jax.ShapeDtypeStruct((NC, ACC_ROWS, H), jnp.float32),
        mesh=mesh,
        scratch_types=[
            pltpu.VMEM((2 * (NCHUNK // NSTAGE), CHUNK), jnp.int32),
            pltpu.VMEM((2 * (NCHUNK // NSTAGE), CHUNK), jnp.int32),
            pltpu.VMEM((NBUF, CHUNK, H), jnp.float32),
            pltpu.VMEM_SHARED((ACC_ROWS, H), jnp.float32),
        ] + [pltpu.SemaphoreType.DMA] * (NBUF + 1),
    )
    def k(gidx_hbm, didx_hbm, feat2_hbm, out_hbm,
          gidx_v, didx_v, rows_v, acc, *sems):
        c = lax.axis_index("c")
        s = lax.axis_index("s")
        HALF = NCHUNK // NSTAGE
        sem_idx = sems[NBUF]

        def stage_copies(hh, sl):
            return (
                pltpu.make_async_copy(
                    gidx_hbm.at[c, s, pl.ds(hh * HALF, HALF)],
                    gidx_v.at[pl.ds(sl * HALF, HALF)], sem_idx),
                pltpu.make_async_copy(
                    didx_hbm.at[s, pl.ds(hh * HALF, HALF)],
                    didx_v.at[pl.ds(sl * HALF, HALF)], sem_idx),
            )

        for cp in stage_copies(0, 0):
            cp.start()

        @plsc.parallel_loop(0, CHUNK, 1)
        def _(r):
            for kk in range(H // 16):
                rows_v[0, r, pl.ds(kk * 16, 16)] = jnp.zeros(
                    (16,), jnp.float32)

        for z in range(RPT // CHUNK):
            pltpu.sync_copy(rows_v.at[0],
                            acc.at[pl.ds(s * RPT + z * CHUNK, CHUNK)])
        plsc.subcore_barrier()

        def gather(j, b, sl):
            return pltpu.make_async_copy(
                feat2_hbm.at[gidx_v.at[sl * HALF + j]], rows_v.at[b], sems[b])

        for cp in stage_copies(0, 0):
            cp.wait()
        for hh in range(NSTAGE):
            sl = hh % 2
            if hh + 1 < NSTAGE:
                for cp in stage_copies(hh + 1, 1 - sl):
                    cp.start()
            for b in range(NBUF):
                gather(b, b, sl).start()

            def body(i, _):
                for b in range(NBUF):
                    j = NBUF * i + b
                    gather(j, b, sl).wait()
                    pltpu.sync_copy(rows_v.at[b],
                                    acc.at[didx_v.at[sl * HALF + j]],
                                    add=True)
                    jn = j + NBUF

                    @pl.when(jn < HALF)
                    def _():
                        gather(jn, b, sl).start()
                return 0

            lax.fori_loop(0, HALF // NBUF, body, 0)
            if hh + 1 < NSTAGE:
                for cp in stage_copies(hh + 1, 1 - sl):
                    cp.wait()
        plsc.subcore_barrier()
        pltpu.sync_copy(acc.at[pl.ds(s * RPT, RPT)],
                        out_hbm.at[c, pl.ds(s * RPT, RPT)])

    return k(gidx, didx, feat2)


def _matmul_kernel(h_ref, w_ref, b_ref, out_ref):
    h0 = h_ref[0]
    h1 = h_ref[1]
    acc = jnp.dot(h0, w_ref[:H, :], preferred_element_type=jnp.float32)
    acc += jnp.dot(h1, w_ref[H:, :], preferred_element_type=jnp.float32)
    out_ref[...] = acc + b_ref[...]


def _tc_linear(h2, W, b):
    R = 2000
    return pl.pallas_call(
        _matmul_kernel,
        grid=(N_NODES // R,),
        in_specs=[
            pl.BlockSpec((NC, R, H), lambda i: (0, i, 0)),
            pl.BlockSpec((D_FEAT, D_OUT), lambda i: (0, 0)),
            pl.BlockSpec((1, D_OUT), lambda i: (0, 0)),
        ],
        out_specs=pl.BlockSpec((R, D_OUT), lambda i: (i, 0)),
        out_shape=jax.ShapeDtypeStruct((N_NODES, D_OUT), jnp.float32),
    )(h2, W, b.reshape(1, D_OUT))


def kernel(feat, edge_index, W, b):
    src = edge_index[0].astype(jnp.int32)
    dst = edge_index[1].astype(jnp.int32)
    src_p = jnp.zeros((EP,), jnp.int32).at[:N_EDGES].set(src)
    dst_p = jnp.full((EP,), N_NODES, jnp.int32).at[:N_EDGES].set(dst)
    gidx = (2 * src_p)[None, :] + jnp.arange(NC, dtype=jnp.int32)[:, None]
    gidx = gidx.reshape(NC, NS, NCHUNK, CHUNK)
    didx = dst_p.reshape(NS, NCHUNK, CHUNK)
    feat2 = feat.reshape(2 * N_NODES, H)

    h2 = _sc_segment_sum(gidx, didx, feat2)
    return _tc_linear(h2, W, b)
